# Initial kernel scaffold; baseline (speedup 1.0000x reference)
import jax
import jax.numpy as jnp
from jax.experimental import pallas as pl


def kernel(h, cand_ids, W):
    raise NotImplementedError("write your pallas kernel here")


# speedup vs baseline: 1.0587x; 1.0587x over previous
import functools

import jax
import jax.numpy as jnp
from jax import lax
from jax.experimental import pallas as pl
from jax.experimental.pallas import tpu as pltpu
from jax.experimental.pallas import tpu_sc as plsc

NC = 2
NS = 16
NW = NC * NS
L = 16


@functools.lru_cache(maxsize=None)
def _build(B, D, K, N):
    assert B % NW == 0, B
    assert D % L == 0, D
    assert K % 2 == 0 and (K // 2) <= 128 and K % 8 == 0 and K >= L, K
    bpw = B // NW
    kh = K // 2
    ngroups = (K + L - 1) // L
    kpad = ngroups * L
    ndc = D // L

    mesh = plsc.VectorSubcoreMesh(core_axis_name="c", subcore_axis_name="s")

    @functools.partial(
        pl.kernel,
        mesh=mesh,
        compiler_params=pltpu.CompilerParams(use_tc_tiling_on_sc=False),
        out_type=jax.ShapeDtypeStruct((B, kpad), jnp.float32),
        scratch_types=[
            pltpu.VMEM((bpw, 2, kh), jnp.int32),
            pltpu.VMEM((bpw, D), jnp.float32),
            pltpu.VMEM((2, kpad, D), jnp.float32),
            pltpu.VMEM((bpw, kpad), jnp.float32),
            pltpu.SemaphoreType.DMA,
            pltpu.SemaphoreType.DMA,
        ],
    )
    def sc_kernel(h_hbm, ids_hbm, w_hbm, out_hbm, idx_v, h_v, emb, out_v,
                  sem0, sem1):
        wid = lax.axis_index("s") * NC + lax.axis_index("c")
        base = wid * bpw

        pltpu.sync_copy(ids_hbm.at[pl.ds(base, bpw)], idx_v)
        pltpu.sync_copy(h_hbm.at[pl.ds(base, bpw)], h_v)

        sems = (sem0, sem1)

        def fire(b, slot):
            pltpu.async_copy(w_hbm.at[idx_v.at[b, 0]],
                             emb.at[slot, pl.ds(0, kh)], sems[slot])
            pltpu.async_copy(w_hbm.at[idx_v.at[b, 1]],
                             emb.at[slot, pl.ds(kh, kh)], sems[slot])

        def drain(slot):
            pltpu.make_async_copy(w_hbm.at[pl.ds(0, K)],
                                  emb.at[slot, pl.ds(0, K)],
                                  sems[slot]).wait()

        lane = lax.iota(jnp.int32, L)
        perms = [lane ^ (1 << i) for i in range(4)]

        dnums = lax.GatherDimensionNumbers(
            offset_dims=(), collapsed_slice_dims=(0,), start_index_map=(0,))

        def shuffle(x, perm):
            return lax.gather(x, perm[:, None], dimension_numbers=dnums,
                              slice_sizes=(1,),
                              mode=lax.GatherScatterMode.PROMISE_IN_BOUNDS)

        def hsum(p):
            for i in range(4):
                p = p + shuffle(p, perms[i])
            return p

        def compute_row(b, slot):
            hc = [h_v[b, pl.ds(c * L, L)] for c in range(ndc)]

            def group(g, carry):
                kb = pl.multiple_of(g * L, L)
                acc = jnp.zeros((L,), jnp.float32)
                for j in range(L):
                    p = hc[0] * emb[slot, kb + j, pl.ds(0, L)]
                    for c in range(1, ndc):
                        p = p + hc[c] * emb[slot, kb + j, pl.ds(c * L, L)]
                    acc = jnp.where(lane == j, hsum(p), acc)
                out_v[b, pl.ds(kb, L)] = acc
                return carry

            lax.fori_loop(0, ngroups, group, 0)

        fire(0, 0)

        def body(t, carry):
            b = 2 * t
            fire(b + 1, 1)
            drain(0)
            compute_row(b, 0)

            @pl.when(t < bpw // 2 - 1)
            def _():
                fire(b + 2, 0)

            drain(1)
            compute_row(b + 1, 1)
            return carry

        lax.fori_loop(0, bpw // 2, body, 0)

        pltpu.sync_copy(out_v, out_hbm.at[pl.ds(base, bpw)])

    return sc_kernel


def kernel(h, cand_ids, W):
    B, D = h.shape
    K = cand_ids.shape[1]
    N = W.shape[0]
    ids3 = cand_ids.astype(jnp.int32).reshape(B, 2, K // 2)
    return _build(B, D, K, N)(h, ids3, W)[:, :K]
